# baseline (speedup 1.0000x reference)
import jax
import jax.numpy as jnp
from jax import lax
from jax.experimental import pallas as pl
from jax.experimental.pallas import tpu as pltpu
from jax.experimental.pallas import tpu_sc as plsc

N = 50000
DIM = 32
G = 1024
OUT = 128
FP = 96

NC, NS = 2, 16
NW = NC * NS
EC = 128
KI = 10

RPT = 1568
RPC = NS * RPT
ACC_ROWS = RPC + EC
ZPS = ACC_ROWS // NS
ZCHUNK = ZPS // 8

BLK = 2000
NB = N // BLK
BLKF = 1000
NBF = N // BLKF


def _bin_edges(edge_index):
    src, dst = edge_index[0], edge_index[1]
    e = src.shape[0]
    epb = ((e + NW * (EC - 1) + EC - 1) // EC) * EC
    ch = epb // EC

    bucket = dst // RPT
    perm = jnp.argsort(bucket, stable=True)
    b_s = bucket[perm]
    starts = jnp.searchsorted(b_s, jnp.arange(NW, dtype=jnp.int32), 'left')
    ends = jnp.searchsorted(b_s, jnp.arange(NW, dtype=jnp.int32), 'right')
    cnt_pad = ((ends - starts + EC - 1) // EC) * EC
    off_pad = jnp.concatenate(
        [jnp.zeros((1,), jnp.int32), jnp.cumsum(cnt_pad)[:-1]]).astype(jnp.int32)
    pos = off_pad[b_s] + (jnp.arange(e, dtype=jnp.int32) - starts[b_s])

    buf = epb + KI * EC
    src2 = jnp.zeros((buf,), jnp.int32).at[pos].set(src[perm])
    dstl2 = jnp.full((buf,), RPC, jnp.int32).at[pos].set(
        dst[perm] - (b_s // NS) * RPC)
    lo = off_pad // EC
    hi = (off_pad + cnt_pad) // EC
    bounds = jnp.pad(jnp.stack([lo, hi], axis=1), ((0, 0), (0, 14)))
    return (src2.reshape(buf // EC, EC), dstl2.reshape(buf // EC, EC),
            bounds, ch)


def _sc_scatter(h, src_w, dstl_w, bounds):
    mesh = plsc.VectorSubcoreMesh(core_axis_name="c", subcore_axis_name="s",
                                  num_cores=NC, num_subcores=NS)

    def body(h_hbm, src_hbm, dstl_hbm, bounds_hbm, out_hbm, acc_sh, src_v,
             dst_v, rowbuf, zbuf, bvec, sem):
        c = lax.axis_index("c")
        s = lax.axis_index("s")
        w = c * NS + s
        zero16 = jnp.zeros((16,), jnp.float32)

        def zrow(i, carry):
            zbuf[i, pl.ds(0, 16)] = zero16
            zbuf[i, pl.ds(16, 16)] = zero16
            return carry
        lax.fori_loop(0, ZCHUNK, zrow, 0)

        def zdma(i, carry):
            pltpu.sync_copy(zbuf,
                            acc_sh.at[pl.ds(s * ZPS + i * ZCHUNK, ZCHUNK), :])
            return carry
        lax.fori_loop(0, 8, zdma, 0)

        pltpu.sync_copy(bounds_hbm.at[w], bvec)
        bv = bvec[...]
        lo = bv[0]
        hi = bv[1]
        plsc.subcore_barrier()

        def group(g, carry):
            start = lo + g * KI
            pltpu.sync_copy(src_hbm.at[pl.ds(start, KI)], src_v)
            pltpu.sync_copy(dstl_hbm.at[pl.ds(start, KI)], dst_v)
            m = jnp.minimum(KI, hi - start)

            def edge(j, carry2):
                pltpu.async_copy(h_hbm.at[src_v.at[j]], rowbuf, sem).wait()
                pltpu.sync_copy(rowbuf, acc_sh.at[dst_v.at[j]], add=True)
                return carry2
            lax.fori_loop(0, m, edge, 0)
            return carry
        lax.fori_loop(0, (hi - lo + KI - 1) // KI, group, 0)
        plsc.subcore_barrier()

        pltpu.sync_copy(acc_sh.at[pl.ds(s * RPT, RPT), :],
                        out_hbm.at[pl.ds(c * RPC + s * RPT, RPT), :])

    f = pl.kernel(
        body,
        out_type=jax.ShapeDtypeStruct((NW * RPT, DIM), jnp.float32),
        mesh=mesh,
        scratch_types=[
            pltpu.VMEM_SHARED((ACC_ROWS, DIM), jnp.float32),
            pltpu.VMEM((KI, EC), jnp.int32),
            pltpu.VMEM((KI, EC), jnp.int32),
            pltpu.VMEM((EC, DIM), jnp.float32),
            pltpu.VMEM((ZCHUNK, DIM), jnp.float32),
            pltpu.VMEM((16,), jnp.int32),
            pltpu.SemaphoreType.DMA,
        ],
        compiler_params=pltpu.CompilerParams(use_tc_tiling_on_sc=False),
    )
    return f(h, src_w, dstl_w, bounds)


def _colsum(r):
    m, d = r.shape
    p2 = 1
    while p2 < m:
        p2 *= 2
    if p2 != m:
        r = jnp.concatenate([r, jnp.zeros((p2 - m, d), r.dtype)], axis=0)
    while p2 > 1:
        half = p2 // 2
        r = r[:half] + r[half:]
        p2 = half
    return r


def _mlp(h_blk, agg_blk, w1_ref, b1_ref, w2_ref, b2_ref):
    z = h_blk + agg_blk
    z1 = jnp.maximum(jnp.dot(z, w1_ref[...],
                             preferred_element_type=jnp.float32)
                     + b1_ref[...], 0.0)
    r = jnp.maximum(jnp.dot(z1, w2_ref[...],
                            preferred_element_type=jnp.float32)
                    + b2_ref[...], 0.0)
    return r


def _tc_layer(h, aggs, w1, b1, w2, b2, gamma, beta):
    f_in = h.shape[1]
    nch = len(aggs)

    def body(*refs):
        h_ref = refs[0]
        agg_refs = refs[1:1 + nch]
        b1_ref, w2_ref, b2_ref, g_ref, be_ref, w1_ref = refs[1 + nch:7 + nch]
        o_ref, r_scr, st_scr = refs[7 + nch:]
        p = pl.program_id(0)
        j = pl.program_id(1)

        @pl.when(p == 0)
        def _phase0():
            agg = (agg_refs[0][...] if nch == 1 else
                   jnp.concatenate([a[...] for a in agg_refs], axis=1))
            r = _mlp(h_ref[...], agg, w1_ref, b1_ref, w2_ref, b2_ref)
            r_scr[pl.ds(j * BLK, BLK), :] = r

            @pl.when(j == 0)
            def _init():
                st_scr[...] = jnp.zeros_like(st_scr)
            st_scr[0:1, :] += _colsum(r)

        @pl.when(p == 1)
        def _phase1():
            mu = st_scr[0:1, :] * (1.0 / N)
            d = r_scr[pl.ds(j * BLK, BLK), :] - mu
            st_scr[1:2, :] += _colsum(d * d)

        @pl.when(p == 2)
        def _phase2():
            mu = st_scr[0:1, :] * (1.0 / N)
            var = st_scr[1:2, :] * (1.0 / N)
            sd = jnp.sqrt(var + 1e-5)
            r = r_scr[pl.ds(j * BLK, BLK), :]
            o_ref[...] = g_ref[...] * (r - mu) / sd + be_ref[...]

    full = lambda shape: pl.BlockSpec(shape, lambda p, j: tuple(0 for _ in shape))
    return pl.pallas_call(
        body,
        grid=(3, NB),
        in_specs=[pl.BlockSpec((BLK, f_in),
                               lambda p, j: (jnp.where(p == 0, j, NB - 1), 0))]
        + [pl.BlockSpec((BLK, DIM),
                        lambda p, j: (jnp.where(p == 0, j, NB - 1), 0))
           for _ in range(nch)]
        + [full((1, DIM)), full((DIM, DIM)), full((1, DIM)),
           full((1, DIM)), full((1, DIM)), full((f_in, DIM))],
        out_specs=pl.BlockSpec((BLK, DIM),
                               lambda p, j: (jnp.where(p == 2, j, 0), 0)),
        out_shape=jax.ShapeDtypeStruct((N, DIM), jnp.float32),
        scratch_shapes=[pltpu.VMEM((N, DIM), jnp.float32),
                        pltpu.VMEM((2, DIM), jnp.float32)],
        compiler_params=pltpu.CompilerParams(
            dimension_semantics=("arbitrary", "arbitrary")),
    )(h, *aggs, b1, w2, b2, gamma, beta, w1)


def _tc_final(h, agg, w1, b1, w2, b2, gamma, beta, batch3, wfc, bfc):
    def body(h_ref, agg_ref, b1_ref, w2_ref, b2_ref, g_ref, be_ref, bt_ref,
             wfc_ref, bfc_ref, w1_ref, o_ref, r_scr, pool_scr, cnt_scr,
             st_scr):
        p = pl.program_id(0)
        j = pl.program_id(1)

        @pl.when(p == 0)
        def _phase0():
            r = _mlp(h_ref[...], agg_ref[...], w1_ref, b1_ref, w2_ref,
                     b2_ref)
            r_scr[pl.ds(j * BLKF, BLKF), :] = r

            @pl.when(j == 0)
            def _init():
                st_scr[...] = jnp.zeros_like(st_scr)
                pool_scr[...] = jnp.zeros_like(pool_scr)
                cnt_scr[...] = jnp.zeros_like(cnt_scr)

            st_scr[0:1, :] += _colsum(r)
            ids = bt_ref[0, 0, :]
            seg = lax.broadcasted_iota(jnp.int32, (G, BLKF), 0)
            oh = (seg == ids[None, :]).astype(jnp.float32)
            pool_scr[...] += jnp.dot(oh, r,
                                     preferred_element_type=jnp.float32,
                                     precision=lax.Precision.HIGHEST)
            cnt_scr[...] += jnp.sum(oh, axis=1, keepdims=True)

        @pl.when(p == 1)
        def _phase1():
            mu = st_scr[0:1, :] * (1.0 / N)
            d = r_scr[pl.ds(j * BLKF, BLKF), :] - mu
            st_scr[1:2, :] += _colsum(d * d)

            @pl.when(j == NBF - 1)
            def _finish():
                var = st_scr[1:2, :] * (1.0 / N)
                a = g_ref[...] / jnp.sqrt(var + 1e-5)
                shift = be_ref[...] - mu * a
                pooled = pool_scr[...] * a + cnt_scr[...] * shift
                o_ref[...] = jnp.maximum(
                    jnp.dot(pooled, wfc_ref[...],
                            preferred_element_type=jnp.float32)
                    + bfc_ref[...], 0.0)

    full = lambda shape: pl.BlockSpec(shape, lambda p, j: tuple(0 for _ in shape))
    return pl.pallas_call(
        body,
        grid=(2, NBF),
        in_specs=[
            pl.BlockSpec((BLKF, DIM),
                         lambda p, j: (jnp.where(p == 0, j, NBF - 1), 0)),
            pl.BlockSpec((BLKF, DIM),
                         lambda p, j: (jnp.where(p == 0, j, NBF - 1), 0)),
            full((1, DIM)), full((DIM, DIM)), full((1, DIM)),
            full((1, DIM)), full((1, DIM)),
            pl.BlockSpec((1, 1, BLKF),
                         lambda p, j: (jnp.where(p == 0, j, NBF - 1), 0, 0)),
            full((DIM, OUT)), full((1, OUT)), full((DIM, DIM)),
        ],
        out_specs=full((G, OUT)),
        out_shape=jax.ShapeDtypeStruct((G, OUT), jnp.float32),
        scratch_shapes=[pltpu.VMEM((N, DIM), jnp.float32),
                        pltpu.VMEM((G, DIM), jnp.float32),
                        pltpu.VMEM((G, 1), jnp.float32),
                        pltpu.VMEM((2, DIM), jnp.float32)],
        compiler_params=pltpu.CompilerParams(
            dimension_semantics=("arbitrary", "arbitrary")),
    )(h, agg, b1, w2, b2, gamma, beta, batch3, wfc, bfc, w1)


def _row(v):
    return v.reshape(1, -1)


def kernel(x, edge_index, batch, params):
    src_w, dstl_w, bounds, _ = _bin_edges(edge_index)
    batch3 = batch.reshape(NBF, 1, BLKF)

    f_in = x.shape[1]
    xp = jnp.pad(x, ((0, 0), (0, FP - f_in)))
    w1p = jnp.pad(params['conv1']['W1'], ((0, FP - f_in), (0, 0)))

    aggs = [_sc_scatter(xp[:, t * DIM:(t + 1) * DIM], src_w, dstl_w, bounds)
            [:N] for t in range(FP // DIM)]
    cp, bp = params['conv1'], params['bn1']
    h = _tc_layer(xp, aggs, w1p, _row(cp['b1']), cp['W2'], _row(cp['b2']),
                  _row(bp['gamma']), _row(bp['beta']))
    for i in range(2, 5):
        agg = _sc_scatter(h, src_w, dstl_w, bounds)[:N]
        cp, bp = params[f'conv{i}'], params[f'bn{i}']
        h = _tc_layer(h, [agg], cp['W1'], _row(cp['b1']), cp['W2'],
                      _row(cp['b2']), _row(bp['gamma']), _row(bp['beta']))
    agg = _sc_scatter(h, src_w, dstl_w, bounds)[:N]
    cp, bp = params['conv5'], params['bn5']
    return _tc_final(h, agg, cp['W1'], _row(cp['b1']), cp['W2'],
                     _row(cp['b2']), _row(bp['gamma']), _row(bp['beta']),
                     batch3, params['fc']['W'], _row(params['fc']['b']))


# speedup vs baseline: 1.0326x; 1.0326x over previous
import jax
import jax.numpy as jnp
from jax import lax
from jax.experimental import pallas as pl
from jax.experimental.pallas import tpu as pltpu
from jax.experimental.pallas import tpu_sc as plsc

N = 50000
DIM = 32
G = 1024
OUT = 128
FP = 96

NC, NS = 2, 16
NW = NC * NS
EC = 128
KI = 10

RPT = 1568
RPC = NS * RPT
ACC_ROWS = RPC + EC
ZPS = ACC_ROWS // NS
ZCHUNK = ZPS // 8

BLK = 2000
NB = N // BLK
BLKF = 1000
NBF = N // BLKF


def _bin_edges(edge_index):
    src, dst = edge_index[0], edge_index[1]
    e = src.shape[0]
    epb = ((e + NW * (EC - 1) + EC - 1) // EC) * EC
    ch = epb // EC

    bucket = dst // RPT
    shift = max(e - 1, 1).bit_length()
    if (NW - 1) << shift < 2**31:
        key = jnp.sort(bucket * (1 << shift) + jnp.arange(e, dtype=jnp.int32))
        perm = key & ((1 << shift) - 1)
        b_s = key >> shift
    else:
        perm = jnp.argsort(bucket, stable=True)
        b_s = bucket[perm]
    starts = jnp.searchsorted(b_s, jnp.arange(NW, dtype=jnp.int32), 'left')
    ends = jnp.searchsorted(b_s, jnp.arange(NW, dtype=jnp.int32), 'right')
    cnt_pad = ((ends - starts + EC - 1) // EC) * EC
    off_pad = jnp.concatenate(
        [jnp.zeros((1,), jnp.int32), jnp.cumsum(cnt_pad)[:-1]]).astype(jnp.int32)
    pos = off_pad[b_s] + (jnp.arange(e, dtype=jnp.int32) - starts[b_s])

    buf = epb + KI * EC
    src2 = jnp.zeros((buf,), jnp.int32).at[pos].set(src[perm])
    dstl2 = jnp.full((buf,), RPC, jnp.int32).at[pos].set(
        dst[perm] - (b_s // NS) * RPC)
    lo = off_pad // EC
    hi = (off_pad + cnt_pad) // EC
    bounds = jnp.pad(jnp.stack([lo, hi], axis=1), ((0, 0), (0, 14)))
    return (src2.reshape(buf // EC, EC), dstl2.reshape(buf // EC, EC),
            bounds, ch)


def _sc_scatter(h, src_w, dstl_w, bounds):
    mesh = plsc.VectorSubcoreMesh(core_axis_name="c", subcore_axis_name="s",
                                  num_cores=NC, num_subcores=NS)

    def body(h_hbm, src_hbm, dstl_hbm, bounds_hbm, out_hbm, acc_sh, src_v,
             dst_v, rowbuf, zbuf, bvec, sem):
        c = lax.axis_index("c")
        s = lax.axis_index("s")
        w = c * NS + s
        zero16 = jnp.zeros((16,), jnp.float32)

        def zrow(i, carry):
            zbuf[i, pl.ds(0, 16)] = zero16
            zbuf[i, pl.ds(16, 16)] = zero16
            return carry
        lax.fori_loop(0, ZCHUNK, zrow, 0)

        def zdma(i, carry):
            pltpu.sync_copy(zbuf,
                            acc_sh.at[pl.ds(s * ZPS + i * ZCHUNK, ZCHUNK), :])
            return carry
        lax.fori_loop(0, 8, zdma, 0)

        pltpu.sync_copy(bounds_hbm.at[w], bvec)
        bv = bvec[...]
        lo = bv[0]
        hi = bv[1]
        plsc.subcore_barrier()

        def group(g, carry):
            start = lo + g * KI
            pltpu.sync_copy(src_hbm.at[pl.ds(start, KI)], src_v)
            pltpu.sync_copy(dstl_hbm.at[pl.ds(start, KI)], dst_v)
            m = jnp.minimum(KI, hi - start)

            def edge(j, carry2):
                pltpu.async_copy(h_hbm.at[src_v.at[j]], rowbuf, sem).wait()
                pltpu.sync_copy(rowbuf, acc_sh.at[dst_v.at[j]], add=True)
                return carry2
            lax.fori_loop(0, m, edge, 0)
            return carry
        lax.fori_loop(0, (hi - lo + KI - 1) // KI, group, 0)
        plsc.subcore_barrier()

        pltpu.sync_copy(acc_sh.at[pl.ds(s * RPT, RPT), :],
                        out_hbm.at[pl.ds(c * RPC + s * RPT, RPT), :])

    f = pl.kernel(
        body,
        out_type=jax.ShapeDtypeStruct((NW * RPT, DIM), jnp.float32),
        mesh=mesh,
        scratch_types=[
            pltpu.VMEM_SHARED((ACC_ROWS, DIM), jnp.float32),
            pltpu.VMEM((KI, EC), jnp.int32),
            pltpu.VMEM((KI, EC), jnp.int32),
            pltpu.VMEM((EC, DIM), jnp.float32),
            pltpu.VMEM((ZCHUNK, DIM), jnp.float32),
            pltpu.VMEM((16,), jnp.int32),
            pltpu.SemaphoreType.DMA,
        ],
        compiler_params=pltpu.CompilerParams(use_tc_tiling_on_sc=False),
    )
    return f(h, src_w, dstl_w, bounds)


def _colsum(r):
    m, d = r.shape
    p2 = 1
    while p2 < m:
        p2 *= 2
    if p2 != m:
        r = jnp.concatenate([r, jnp.zeros((p2 - m, d), r.dtype)], axis=0)
    while p2 > 1:
        half = p2 // 2
        r = r[:half] + r[half:]
        p2 = half
    return r


def _mlp(h_blk, agg_blk, w1_ref, b1_ref, w2_ref, b2_ref):
    z = h_blk + agg_blk
    z1 = jnp.maximum(jnp.dot(z, w1_ref[...],
                             preferred_element_type=jnp.float32)
                     + b1_ref[...], 0.0)
    r = jnp.maximum(jnp.dot(z1, w2_ref[...],
                            preferred_element_type=jnp.float32)
                    + b2_ref[...], 0.0)
    return r


def _tc_layer(h, aggs, w1, b1, w2, b2, gamma, beta):
    f_in = h.shape[1]
    nch = len(aggs)

    def body(*refs):
        h_ref = refs[0]
        agg_refs = refs[1:1 + nch]
        b1_ref, w2_ref, b2_ref, g_ref, be_ref, w1_ref = refs[1 + nch:7 + nch]
        o_ref, r_scr, st_scr = refs[7 + nch:]
        p = pl.program_id(0)
        j = pl.program_id(1)

        @pl.when(p == 0)
        def _phase0():
            agg = (agg_refs[0][...] if nch == 1 else
                   jnp.concatenate([a[...] for a in agg_refs], axis=1))
            r = _mlp(h_ref[...], agg, w1_ref, b1_ref, w2_ref, b2_ref)
            r_scr[pl.ds(j * BLK, BLK), :] = r

            @pl.when(j == 0)
            def _init():
                st_scr[...] = jnp.zeros_like(st_scr)
            st_scr[0:1, :] += _colsum(r)

        @pl.when(p == 1)
        def _phase1():
            mu = st_scr[0:1, :] * (1.0 / N)
            d = r_scr[pl.ds(j * BLK, BLK), :] - mu
            st_scr[1:2, :] += _colsum(d * d)

        @pl.when(p == 2)
        def _phase2():
            mu = st_scr[0:1, :] * (1.0 / N)
            var = st_scr[1:2, :] * (1.0 / N)
            sd = jnp.sqrt(var + 1e-5)
            r = r_scr[pl.ds(j * BLK, BLK), :]
            o_ref[...] = g_ref[...] * (r - mu) / sd + be_ref[...]

    full = lambda shape: pl.BlockSpec(shape, lambda p, j: tuple(0 for _ in shape))
    return pl.pallas_call(
        body,
        grid=(3, NB),
        in_specs=[pl.BlockSpec((BLK, f_in),
                               lambda p, j: (jnp.where(p == 0, j, NB - 1), 0))]
        + [pl.BlockSpec((BLK, DIM),
                        lambda p, j: (jnp.where(p == 0, j, NB - 1), 0))
           for _ in range(nch)]
        + [full((1, DIM)), full((DIM, DIM)), full((1, DIM)),
           full((1, DIM)), full((1, DIM)), full((f_in, DIM))],
        out_specs=pl.BlockSpec((BLK, DIM),
                               lambda p, j: (jnp.where(p == 2, j, 0), 0)),
        out_shape=jax.ShapeDtypeStruct((N, DIM), jnp.float32),
        scratch_shapes=[pltpu.VMEM((N, DIM), jnp.float32),
                        pltpu.VMEM((2, DIM), jnp.float32)],
        compiler_params=pltpu.CompilerParams(
            dimension_semantics=("arbitrary", "arbitrary")),
    )(h, *aggs, b1, w2, b2, gamma, beta, w1)


def _tc_final(h, agg, w1, b1, w2, b2, gamma, beta, batch3, wfc, bfc):
    def body(h_ref, agg_ref, b1_ref, w2_ref, b2_ref, g_ref, be_ref, bt_ref,
             wfc_ref, bfc_ref, w1_ref, o_ref, r_scr, pool_scr, cnt_scr,
             st_scr):
        p = pl.program_id(0)
        j = pl.program_id(1)

        @pl.when(p == 0)
        def _phase0():
            r = _mlp(h_ref[...], agg_ref[...], w1_ref, b1_ref, w2_ref,
                     b2_ref)
            r_scr[pl.ds(j * BLKF, BLKF), :] = r

            @pl.when(j == 0)
            def _init():
                st_scr[...] = jnp.zeros_like(st_scr)
                pool_scr[...] = jnp.zeros_like(pool_scr)
                cnt_scr[...] = jnp.zeros_like(cnt_scr)

            st_scr[0:1, :] += _colsum(r)
            ids = bt_ref[0, 0, :]
            seg = lax.broadcasted_iota(jnp.int32, (G, BLKF), 0)
            oh = (seg == ids[None, :]).astype(jnp.float32)
            pool_scr[...] += jnp.dot(oh, r,
                                     preferred_element_type=jnp.float32,
                                     precision=lax.Precision.HIGHEST)
            cnt_scr[...] += jnp.sum(oh, axis=1, keepdims=True)

        @pl.when(p == 1)
        def _phase1():
            mu = st_scr[0:1, :] * (1.0 / N)
            d = r_scr[pl.ds(j * BLKF, BLKF), :] - mu
            st_scr[1:2, :] += _colsum(d * d)

            @pl.when(j == NBF - 1)
            def _finish():
                var = st_scr[1:2, :] * (1.0 / N)
                a = g_ref[...] / jnp.sqrt(var + 1e-5)
                shift = be_ref[...] - mu * a
                pooled = pool_scr[...] * a + cnt_scr[...] * shift
                o_ref[...] = jnp.maximum(
                    jnp.dot(pooled, wfc_ref[...],
                            preferred_element_type=jnp.float32)
                    + bfc_ref[...], 0.0)

    full = lambda shape: pl.BlockSpec(shape, lambda p, j: tuple(0 for _ in shape))
    return pl.pallas_call(
        body,
        grid=(2, NBF),
        in_specs=[
            pl.BlockSpec((BLKF, DIM),
                         lambda p, j: (jnp.where(p == 0, j, NBF - 1), 0)),
            pl.BlockSpec((BLKF, DIM),
                         lambda p, j: (jnp.where(p == 0, j, NBF - 1), 0)),
            full((1, DIM)), full((DIM, DIM)), full((1, DIM)),
            full((1, DIM)), full((1, DIM)),
            pl.BlockSpec((1, 1, BLKF),
                         lambda p, j: (jnp.where(p == 0, j, NBF - 1), 0, 0)),
            full((DIM, OUT)), full((1, OUT)), full((DIM, DIM)),
        ],
        out_specs=full((G, OUT)),
        out_shape=jax.ShapeDtypeStruct((G, OUT), jnp.float32),
        scratch_shapes=[pltpu.VMEM((N, DIM), jnp.float32),
                        pltpu.VMEM((G, DIM), jnp.float32),
                        pltpu.VMEM((G, 1), jnp.float32),
                        pltpu.VMEM((2, DIM), jnp.float32)],
        compiler_params=pltpu.CompilerParams(
            dimension_semantics=("arbitrary", "arbitrary")),
    )(h, agg, b1, w2, b2, gamma, beta, batch3, wfc, bfc, w1)


def _row(v):
    return v.reshape(1, -1)


def kernel(x, edge_index, batch, params):
    src_w, dstl_w, bounds, _ = _bin_edges(edge_index)
    batch3 = batch.reshape(NBF, 1, BLKF)

    f_in = x.shape[1]
    xp = jnp.pad(x, ((0, 0), (0, FP - f_in)))
    w1p = jnp.pad(params['conv1']['W1'], ((0, FP - f_in), (0, 0)))

    aggs = [_sc_scatter(xp[:, t * DIM:(t + 1) * DIM], src_w, dstl_w, bounds)
            [:N] for t in range(FP // DIM)]
    cp, bp = params['conv1'], params['bn1']
    h = _tc_layer(xp, aggs, w1p, _row(cp['b1']), cp['W2'], _row(cp['b2']),
                  _row(bp['gamma']), _row(bp['beta']))
    for i in range(2, 5):
        agg = _sc_scatter(h, src_w, dstl_w, bounds)[:N]
        cp, bp = params[f'conv{i}'], params[f'bn{i}']
        h = _tc_layer(h, [agg], cp['W1'], _row(cp['b1']), cp['W2'],
                      _row(cp['b2']), _row(bp['gamma']), _row(bp['beta']))
    agg = _sc_scatter(h, src_w, dstl_w, bounds)[:N]
    cp, bp = params['conv5'], params['bn5']
    return _tc_final(h, agg, cp['W1'], _row(cp['b1']), cp['W2'],
                     _row(cp['b2']), _row(bp['gamma']), _row(bp['beta']),
                     batch3, params['fc']['W'], _row(params['fc']['b']))


# speedup vs baseline: 2.2501x; 2.1790x over previous
import jax
import jax.numpy as jnp
from jax import lax
from jax.experimental import pallas as pl
from jax.experimental.pallas import tpu as pltpu
from jax.experimental.pallas import tpu_sc as plsc

N = 50000
DIM = 32
G = 1024
OUT = 128
FP = 96

NC, NS = 2, 16
NW = NC * NS
EC = 128
KI = 10

RPT = 1568
RPC = NS * RPT
ACC_ROWS = RPC + EC
ZPS = ACC_ROWS // NS
ZCHUNK = ZPS // 8

BLK = 2000
NB = N // BLK
BLKF = 1000
NBF = N // BLKF


def _bin_edges(edge_index):
    src, dst = edge_index[0], edge_index[1]
    e = src.shape[0]
    epb = ((e + NW * (EC - 1) + EC - 1) // EC) * EC
    ch = epb // EC

    bucket = dst // RPT
    shift = max(e - 1, 1).bit_length()
    if (NW - 1) << shift < 2**31:
        key = jnp.sort(bucket * (1 << shift) + jnp.arange(e, dtype=jnp.int32))
        perm = key & ((1 << shift) - 1)
        b_s = key >> shift
    else:
        perm = jnp.argsort(bucket, stable=True)
        b_s = bucket[perm]
    starts = jnp.searchsorted(b_s, jnp.arange(NW, dtype=jnp.int32), 'left')
    ends = jnp.searchsorted(b_s, jnp.arange(NW, dtype=jnp.int32), 'right')
    cnt = ends - starts
    cnt_pad = ((cnt + EC - 1) // EC) * EC
    off_pad = jnp.concatenate(
        [jnp.zeros((1,), jnp.int32), jnp.cumsum(cnt_pad)[:-1]]).astype(jnp.int32)

    buf = epb + KI * EC
    parr = jnp.arange(buf, dtype=jnp.int32)
    wp = (jnp.searchsorted(off_pad, parr, side='right') - 1).astype(jnp.int32)
    rank = parr - off_pad[wp]
    valid = rank < cnt[wp]
    eidx = perm[starts[wp] + jnp.where(valid, rank, 0)]
    src2 = jnp.where(valid, src[eidx], 0)
    dstl2 = jnp.where(valid, dst[eidx] - (wp // NS) * RPC, RPC)
    lo = off_pad // EC
    hi = (off_pad + cnt_pad) // EC
    bounds = jnp.pad(jnp.stack([lo, hi], axis=1), ((0, 0), (0, 14)))
    return (src2.reshape(buf // EC, EC), dstl2.reshape(buf // EC, EC),
            bounds, ch)


def _sc_scatter(h, src_w, dstl_w, bounds):
    mesh = plsc.VectorSubcoreMesh(core_axis_name="c", subcore_axis_name="s",
                                  num_cores=NC, num_subcores=NS)

    def body(h_hbm, src_hbm, dstl_hbm, bounds_hbm, out_hbm, acc_sh, src_v,
             dst_v, rowbuf, zbuf, bvec, sem):
        c = lax.axis_index("c")
        s = lax.axis_index("s")
        w = c * NS + s
        zero16 = jnp.zeros((16,), jnp.float32)

        def zrow(i, carry):
            zbuf[i, pl.ds(0, 16)] = zero16
            zbuf[i, pl.ds(16, 16)] = zero16
            return carry
        lax.fori_loop(0, ZCHUNK, zrow, 0)

        def zdma(i, carry):
            pltpu.sync_copy(zbuf,
                            acc_sh.at[pl.ds(s * ZPS + i * ZCHUNK, ZCHUNK), :])
            return carry
        lax.fori_loop(0, 8, zdma, 0)

        pltpu.sync_copy(bounds_hbm.at[w], bvec)
        bv = bvec[...]
        lo = bv[0]
        hi = bv[1]
        plsc.subcore_barrier()

        def group(g, carry):
            start = lo + g * KI
            pltpu.sync_copy(src_hbm.at[pl.ds(start, KI)], src_v)
            pltpu.sync_copy(dstl_hbm.at[pl.ds(start, KI)], dst_v)
            m = jnp.minimum(KI, hi - start)

            def edge(j, carry2):
                pltpu.async_copy(h_hbm.at[src_v.at[j]], rowbuf, sem).wait()
                pltpu.sync_copy(rowbuf, acc_sh.at[dst_v.at[j]], add=True)
                return carry2
            lax.fori_loop(0, m, edge, 0)
            return carry
        lax.fori_loop(0, (hi - lo + KI - 1) // KI, group, 0)
        plsc.subcore_barrier()

        pltpu.sync_copy(acc_sh.at[pl.ds(s * RPT, RPT), :],
                        out_hbm.at[pl.ds(c * RPC + s * RPT, RPT), :])

    f = pl.kernel(
        body,
        out_type=jax.ShapeDtypeStruct((NW * RPT, DIM), jnp.float32),
        mesh=mesh,
        scratch_types=[
            pltpu.VMEM_SHARED((ACC_ROWS, DIM), jnp.float32),
            pltpu.VMEM((KI, EC), jnp.int32),
            pltpu.VMEM((KI, EC), jnp.int32),
            pltpu.VMEM((EC, DIM), jnp.float32),
            pltpu.VMEM((ZCHUNK, DIM), jnp.float32),
            pltpu.VMEM((16,), jnp.int32),
            pltpu.SemaphoreType.DMA,
        ],
        compiler_params=pltpu.CompilerParams(use_tc_tiling_on_sc=False),
    )
    return f(h, src_w, dstl_w, bounds)


def _colsum(r):
    m, d = r.shape
    p2 = 1
    while p2 < m:
        p2 *= 2
    if p2 != m:
        r = jnp.concatenate([r, jnp.zeros((p2 - m, d), r.dtype)], axis=0)
    while p2 > 1:
        half = p2 // 2
        r = r[:half] + r[half:]
        p2 = half
    return r


def _mlp(h_blk, agg_blk, w1_ref, b1_ref, w2_ref, b2_ref):
    z = h_blk + agg_blk
    z1 = jnp.maximum(jnp.dot(z, w1_ref[...],
                             preferred_element_type=jnp.float32)
                     + b1_ref[...], 0.0)
    r = jnp.maximum(jnp.dot(z1, w2_ref[...],
                            preferred_element_type=jnp.float32)
                    + b2_ref[...], 0.0)
    return r


def _tc_layer(h, aggs, w1, b1, w2, b2, gamma, beta):
    f_in = h.shape[1]
    nch = len(aggs)

    def body(*refs):
        h_ref = refs[0]
        agg_refs = refs[1:1 + nch]
        b1_ref, w2_ref, b2_ref, g_ref, be_ref, w1_ref = refs[1 + nch:7 + nch]
        o_ref, r_scr, st_scr = refs[7 + nch:]
        p = pl.program_id(0)
        j = pl.program_id(1)

        @pl.when(p == 0)
        def _phase0():
            agg = (agg_refs[0][...] if nch == 1 else
                   jnp.concatenate([a[...] for a in agg_refs], axis=1))
            r = _mlp(h_ref[...], agg, w1_ref, b1_ref, w2_ref, b2_ref)
            r_scr[pl.ds(j * BLK, BLK), :] = r

            @pl.when(j == 0)
            def _init():
                st_scr[...] = jnp.zeros_like(st_scr)
            st_scr[0:1, :] += _colsum(r)

        @pl.when(p == 1)
        def _phase1():
            mu = st_scr[0:1, :] * (1.0 / N)
            d = r_scr[pl.ds(j * BLK, BLK), :] - mu
            st_scr[1:2, :] += _colsum(d * d)

        @pl.when(p == 2)
        def _phase2():
            mu = st_scr[0:1, :] * (1.0 / N)
            var = st_scr[1:2, :] * (1.0 / N)
            sd = jnp.sqrt(var + 1e-5)
            r = r_scr[pl.ds(j * BLK, BLK), :]
            o_ref[...] = g_ref[...] * (r - mu) / sd + be_ref[...]

    full = lambda shape: pl.BlockSpec(shape, lambda p, j: tuple(0 for _ in shape))
    return pl.pallas_call(
        body,
        grid=(3, NB),
        in_specs=[pl.BlockSpec((BLK, f_in),
                               lambda p, j: (jnp.where(p == 0, j, NB - 1), 0))]
        + [pl.BlockSpec((BLK, DIM),
                        lambda p, j: (jnp.where(p == 0, j, NB - 1), 0))
           for _ in range(nch)]
        + [full((1, DIM)), full((DIM, DIM)), full((1, DIM)),
           full((1, DIM)), full((1, DIM)), full((f_in, DIM))],
        out_specs=pl.BlockSpec((BLK, DIM),
                               lambda p, j: (jnp.where(p == 2, j, 0), 0)),
        out_shape=jax.ShapeDtypeStruct((N, DIM), jnp.float32),
        scratch_shapes=[pltpu.VMEM((N, DIM), jnp.float32),
                        pltpu.VMEM((2, DIM), jnp.float32)],
        compiler_params=pltpu.CompilerParams(
            dimension_semantics=("arbitrary", "arbitrary")),
    )(h, *aggs, b1, w2, b2, gamma, beta, w1)


def _tc_final(h, agg, w1, b1, w2, b2, gamma, beta, batch3, wfc, bfc):
    def body(h_ref, agg_ref, b1_ref, w2_ref, b2_ref, g_ref, be_ref, bt_ref,
             wfc_ref, bfc_ref, w1_ref, o_ref, r_scr, pool_scr, cnt_scr,
             st_scr):
        p = pl.program_id(0)
        j = pl.program_id(1)

        @pl.when(p == 0)
        def _phase0():
            r = _mlp(h_ref[...], agg_ref[...], w1_ref, b1_ref, w2_ref,
                     b2_ref)
            r_scr[pl.ds(j * BLKF, BLKF), :] = r

            @pl.when(j == 0)
            def _init():
                st_scr[...] = jnp.zeros_like(st_scr)
                pool_scr[...] = jnp.zeros_like(pool_scr)
                cnt_scr[...] = jnp.zeros_like(cnt_scr)

            st_scr[0:1, :] += _colsum(r)
            ids = bt_ref[0, 0, :]
            seg = lax.broadcasted_iota(jnp.int32, (G, BLKF), 0)
            oh = (seg == ids[None, :]).astype(jnp.float32)
            pool_scr[...] += jnp.dot(oh, r,
                                     preferred_element_type=jnp.float32,
                                     precision=lax.Precision.HIGHEST)
            cnt_scr[...] += jnp.sum(oh, axis=1, keepdims=True)

        @pl.when(p == 1)
        def _phase1():
            mu = st_scr[0:1, :] * (1.0 / N)
            d = r_scr[pl.ds(j * BLKF, BLKF), :] - mu
            st_scr[1:2, :] += _colsum(d * d)

            @pl.when(j == NBF - 1)
            def _finish():
                var = st_scr[1:2, :] * (1.0 / N)
                a = g_ref[...] / jnp.sqrt(var + 1e-5)
                shift = be_ref[...] - mu * a
                pooled = pool_scr[...] * a + cnt_scr[...] * shift
                o_ref[...] = jnp.maximum(
                    jnp.dot(pooled, wfc_ref[...],
                            preferred_element_type=jnp.float32)
                    + bfc_ref[...], 0.0)

    full = lambda shape: pl.BlockSpec(shape, lambda p, j: tuple(0 for _ in shape))
    return pl.pallas_call(
        body,
        grid=(2, NBF),
        in_specs=[
            pl.BlockSpec((BLKF, DIM),
                         lambda p, j: (jnp.where(p == 0, j, NBF - 1), 0)),
            pl.BlockSpec((BLKF, DIM),
                         lambda p, j: (jnp.where(p == 0, j, NBF - 1), 0)),
            full((1, DIM)), full((DIM, DIM)), full((1, DIM)),
            full((1, DIM)), full((1, DIM)),
            pl.BlockSpec((1, 1, BLKF),
                         lambda p, j: (jnp.where(p == 0, j, NBF - 1), 0, 0)),
            full((DIM, OUT)), full((1, OUT)), full((DIM, DIM)),
        ],
        out_specs=full((G, OUT)),
        out_shape=jax.ShapeDtypeStruct((G, OUT), jnp.float32),
        scratch_shapes=[pltpu.VMEM((N, DIM), jnp.float32),
                        pltpu.VMEM((G, DIM), jnp.float32),
                        pltpu.VMEM((G, 1), jnp.float32),
                        pltpu.VMEM((2, DIM), jnp.float32)],
        compiler_params=pltpu.CompilerParams(
            dimension_semantics=("arbitrary", "arbitrary")),
    )(h, agg, b1, w2, b2, gamma, beta, batch3, wfc, bfc, w1)


def _row(v):
    return v.reshape(1, -1)


def kernel(x, edge_index, batch, params):
    src_w, dstl_w, bounds, _ = _bin_edges(edge_index)
    batch3 = batch.reshape(NBF, 1, BLKF)

    f_in = x.shape[1]
    xp = jnp.pad(x, ((0, 0), (0, FP - f_in)))
    w1p = jnp.pad(params['conv1']['W1'], ((0, FP - f_in), (0, 0)))

    aggs = [_sc_scatter(xp[:, t * DIM:(t + 1) * DIM], src_w, dstl_w, bounds)
            [:N] for t in range(FP // DIM)]
    cp, bp = params['conv1'], params['bn1']
    h = _tc_layer(xp, aggs, w1p, _row(cp['b1']), cp['W2'], _row(cp['b2']),
                  _row(bp['gamma']), _row(bp['beta']))
    for i in range(2, 5):
        agg = _sc_scatter(h, src_w, dstl_w, bounds)[:N]
        cp, bp = params[f'conv{i}'], params[f'bn{i}']
        h = _tc_layer(h, [agg], cp['W1'], _row(cp['b1']), cp['W2'],
                      _row(cp['b2']), _row(bp['gamma']), _row(bp['beta']))
    agg = _sc_scatter(h, src_w, dstl_w, bounds)[:N]
    cp, bp = params['conv5'], params['bn5']
    return _tc_final(h, agg, cp['W1'], _row(cp['b1']), cp['W2'],
                     _row(cp['b2']), _row(bp['gamma']), _row(bp['beta']),
                     batch3, params['fc']['W'], _row(params['fc']['b']))
